# baseline (speedup 1.0000x reference)
import jax
import jax.numpy as jnp
from jax.experimental import pallas as pl
from jax.experimental.pallas import tpu as pltpu

_K = 64
_INV_TAU = 10.0
_R = 32
_W = 1_000_000
_CW = 1024
_NC = _W // _CW
_TAIL = _W - _NC * _CW
_CPB = 8
_BK = _CPB * _CW
_NBK = 123
_NP = _NC + _CPB
_MIN32 = -2147483648


def _monotone_key(x):
    b = jax.lax.bitcast_convert_type(x, jnp.int32)
    return jnp.where(b >= 0, b, jnp.int32(_MIN32) - b)


def _chunkmax_body(x_ref, cm_ref):
    c = pl.program_id(1)
    ninf = jnp.full((8, 128 - _CPB), -jnp.inf, jnp.float32)

    @pl.when(c < _NBK - 1)
    def _full():
        mx = [jnp.max(x_ref[:, pl.ds(j * _CW, _CW)], axis=1, keepdims=True)
              for j in range(_CPB)]
        cm_ref[:, 0, 0, :] = jnp.concatenate(mx + [ninf], axis=1)

    @pl.when(c == _NBK - 1)
    def _tail():
        cm_ref[:, 0, 0, :] = jnp.full((8, 128), -jnp.inf, jnp.float32)


def _select_body(cm_ref, perm_ref):
    p_iota = jax.lax.broadcasted_iota(jnp.int32, (_R, _NP), 1)
    valid = p_iota < _NC
    keys = jnp.where(valid, _monotone_key(cm_ref[...]), jnp.int32(_MIN32))
    cnt = jnp.sum((keys >= 0).astype(jnp.int32), axis=1, keepdims=True)
    t = jnp.where(cnt >= _K, jnp.int32(0), jnp.int32(_MIN32))
    for b in range(30, -1, -1):
        cand_t = t + jnp.int32(1 << b)
        cnt = jnp.sum((keys >= cand_t).astype(jnp.int32), axis=1,
                      keepdims=True)
        t = jnp.where(cnt >= _K, cand_t, t)
    tri_r = jax.lax.broadcasted_iota(jnp.int32, (_NP, _NP), 0)
    tri_c = jax.lax.broadcasted_iota(jnp.int32, (_NP, _NP), 1)
    le = (tri_r <= tri_c).astype(jnp.float32)
    above = (keys > t)
    q = jnp.sum(above.astype(jnp.float32), axis=1, keepdims=True)
    tie = (keys == t)
    tie_rank = jnp.dot(tie.astype(jnp.float32), le,
                       preferred_element_type=jnp.float32)
    sel = above | (tie & (tie_rank <= (_K - q)))
    rank = jnp.dot(sel.astype(jnp.float32), le,
                   preferred_element_type=jnp.float32).astype(jnp.int32) - 1
    picked = jnp.where(sel, rank, jnp.int32(-1))
    for s in range(_K):
        perm_ref[:, pl.ds(s, 1)] = jnp.sum(
            jnp.where(picked == s, p_iota, 0), axis=1, keepdims=True)


def _thresh_body(perm_ref, x_ref, tail_ref, v_ref, gbuf_ref, cand_ref,
                 key_ref, sem):
    r = pl.program_id(0)
    rg = r // 8
    rm = jax.lax.rem(r, 8)
    copies = []
    for s in range(_K):
        off = perm_ref[r, s] * _CW
        cp = pltpu.make_async_copy(
            x_ref.at[pl.ds(rg * 8, 8), pl.ds(off, _CW)],
            gbuf_ref.at[pl.ds(s * 8, 8), :], sem)
        cp.start()
        copies.append(cp)
    cand_ref[pl.ds(_K, 8), :] = jnp.full((8, _CW), -jnp.inf, jnp.float32)
    lane = jax.lax.broadcasted_iota(jnp.int32, (1, _CW), 1)
    cand_ref[pl.ds(_K, 1), :] = jnp.where(
        lane < _TAIL, tail_ref[pl.ds(rm, 1), :], -jnp.inf)
    for cp in copies:
        cp.wait()
    for s in range(_K):
        cand_ref[pl.ds(s, 1), :] = gbuf_ref[pl.ds(s * 8 + rm, 1), :]

    key_ref[...] = _monotone_key(cand_ref[...])

    def count_ge(tt):
        return jnp.sum((key_ref[...] >= tt).astype(jnp.int32), axis=(0, 1),
                       keepdims=True)

    t = jnp.where(count_ge(jnp.int32(0)) >= _K, jnp.int32(0),
                  jnp.int32(_MIN32))
    for b in range(30, -1, -1):
        cand_t = t + jnp.int32(1 << b)
        t = jnp.where(count_ge(cand_t) >= _K, cand_t, t)
    v_bits = jnp.where(t >= 0, t, jnp.int32(_MIN32) - t)
    v = jax.lax.bitcast_convert_type(v_bits, jnp.float32)
    v_ref[...] = jnp.broadcast_to(v.reshape(1, 1, 1), (1, 1, 128))


def _gate_body(x_ref, v_ref, o_ref):
    vv = v_ref[:, 0, pl.ds(0, 1)]
    xs = x_ref[...]
    z = (vv - xs) * jnp.float32(_INV_TAU)
    o_ref[...] = xs / (1.0 + jnp.exp(z))


def kernel(logits):
    cm3 = pl.pallas_call(
        _chunkmax_body,
        grid=(_R // 8, _NBK),
        in_specs=[pl.BlockSpec((8, _BK), lambda i, c: (i, c))],
        out_specs=pl.BlockSpec((8, 1, 1, 128), lambda i, c: (i, c, 0, 0)),
        out_shape=jax.ShapeDtypeStruct((_R, _NBK, 1, 128), jnp.float32),
    )(logits)
    cm = cm3[:, :, 0, :_CPB].reshape(_R, _NP)
    perm = pl.pallas_call(
        _select_body,
        out_shape=jax.ShapeDtypeStruct((_R, _K), jnp.int32),
    )(cm)
    v = pl.pallas_call(
        _thresh_body,
        grid=(_R,),
        in_specs=[
            pl.BlockSpec(memory_space=pltpu.SMEM),
            pl.BlockSpec(memory_space=pl.ANY),
            pl.BlockSpec((8, _CW), lambda i: (i // 8, _NC)),
        ],
        out_specs=pl.BlockSpec((1, 1, 128), lambda i: (i, 0, 0)),
        out_shape=jax.ShapeDtypeStruct((_R, 1, 128), jnp.float32),
        scratch_shapes=[
            pltpu.VMEM((8 * _K, _CW), jnp.float32),
            pltpu.VMEM((_K + 8, _CW), jnp.float32),
            pltpu.VMEM((_K + 8, _CW), jnp.int32),
            pltpu.SemaphoreType.DMA,
        ],
    )(perm, logits, logits)
    out = pl.pallas_call(
        _gate_body,
        grid=(_R // 8, _NBK),
        in_specs=[
            pl.BlockSpec((8, _BK), lambda i, c: (i, c)),
            pl.BlockSpec((8, 1, 128), lambda i, c: (i, 0, 0)),
        ],
        out_specs=pl.BlockSpec((8, _BK), lambda i, c: (i, c)),
        out_shape=jax.ShapeDtypeStruct((_R, _W), jnp.float32),
    )(logits, v)
    return out


# speedup vs baseline: 2.1607x; 2.1607x over previous
import jax
import jax.numpy as jnp
from jax.experimental import pallas as pl
from jax.experimental.pallas import tpu as pltpu

_K = 64
_INV_TAU = 10.0
_R = 32
_W = 1_000_000
_CW = 1024
_NC = _W // _CW
_TAIL = _W - _NC * _CW
_CPB = 128
_BK = _CPB * _CW
_NBK = 8
_NP = _NBK * _CPB
_MIN32 = -2147483648


def _monotone_key(x):
    b = jax.lax.bitcast_convert_type(x, jnp.int32)
    return jnp.where(b >= 0, b, jnp.int32(_MIN32) - b)


def _chunkmax_body(x_ref, cm_ref):
    mx = [jnp.max(x_ref[:, pl.ds(j * _CW, _CW)], axis=1, keepdims=True)
          for j in range(_CPB)]
    cm_ref[:, 0, 0, :] = jnp.concatenate(mx, axis=1)


def _select_body(cm_ref, perm_ref):
    p_iota = jax.lax.broadcasted_iota(jnp.int32, (_R, _NP), 1)
    valid = p_iota < _NC
    keys = jnp.where(valid, _monotone_key(cm_ref[...]), jnp.int32(_MIN32))
    cnt = jnp.sum((keys >= 0).astype(jnp.int32), axis=1, keepdims=True)
    t = jnp.where(cnt >= _K, jnp.int32(0), jnp.int32(_MIN32))
    for b in range(30, -1, -1):
        cand_t = t + jnp.int32(1 << b)
        cnt = jnp.sum((keys >= cand_t).astype(jnp.int32), axis=1,
                      keepdims=True)
        t = jnp.where(cnt >= _K, cand_t, t)
    tri_r = jax.lax.broadcasted_iota(jnp.int32, (_NP, _NP), 0)
    tri_c = jax.lax.broadcasted_iota(jnp.int32, (_NP, _NP), 1)
    le = (tri_r <= tri_c).astype(jnp.float32)
    above = (keys > t)
    q = jnp.sum(above.astype(jnp.float32), axis=1, keepdims=True)
    tie = (keys == t)
    tie_rank = jnp.dot(tie.astype(jnp.float32), le,
                       preferred_element_type=jnp.float32)
    sel = above | (tie & (tie_rank <= (_K - q)))
    rank = jnp.dot(sel.astype(jnp.float32), le,
                   preferred_element_type=jnp.float32).astype(jnp.int32) - 1
    picked = jnp.where(sel, rank, jnp.int32(-1))
    for s in range(_K):
        perm_ref[:, pl.ds(s, 1)] = jnp.sum(
            jnp.where(picked == s, p_iota, 0), axis=1, keepdims=True)


def _thresh_body(perm_ref, x_ref, tail_ref, v_ref, gbuf_ref, cand_ref,
                 key_ref, sem):
    r = pl.program_id(0)
    rg = r // 8
    rm = jax.lax.rem(r, 8)
    copies = []
    for s in range(_K):
        off = perm_ref[r, s] * _CW
        cp = pltpu.make_async_copy(
            x_ref.at[pl.ds(rg * 8, 8), pl.ds(off, _CW)],
            gbuf_ref.at[pl.ds(s * 8, 8), :], sem)
        cp.start()
        copies.append(cp)
    cand_ref[pl.ds(_K, 8), :] = jnp.full((8, _CW), -jnp.inf, jnp.float32)
    lane = jax.lax.broadcasted_iota(jnp.int32, (1, _CW), 1)
    cand_ref[pl.ds(_K, 1), :] = jnp.where(
        lane < _TAIL, tail_ref[pl.ds(rm, 1), :], -jnp.inf)
    for cp in copies:
        cp.wait()
    for s in range(_K):
        cand_ref[pl.ds(s, 1), :] = gbuf_ref[pl.ds(s * 8 + rm, 1), :]

    key_ref[...] = _monotone_key(cand_ref[...])

    def count_ge(tt):
        return jnp.sum((key_ref[...] >= tt).astype(jnp.int32), axis=(0, 1),
                       keepdims=True)

    t = jnp.where(count_ge(jnp.int32(0)) >= _K, jnp.int32(0),
                  jnp.int32(_MIN32))
    for b in range(30, -1, -1):
        cand_t = t + jnp.int32(1 << b)
        t = jnp.where(count_ge(cand_t) >= _K, cand_t, t)
    v_bits = jnp.where(t >= 0, t, jnp.int32(_MIN32) - t)
    v = jax.lax.bitcast_convert_type(v_bits, jnp.float32)
    v_ref[...] = jnp.broadcast_to(v.reshape(1, 1, 1), (1, 1, 128))


def _gate_body(x_ref, v_ref, o_ref):
    vv = v_ref[:, 0, pl.ds(0, 1)]
    xs = x_ref[...]
    z = (vv - xs) * jnp.float32(_INV_TAU)
    o_ref[...] = xs / (1.0 + jnp.exp(z))


def kernel(logits):
    cm3 = pl.pallas_call(
        _chunkmax_body,
        grid=(_R // 8, _NBK),
        in_specs=[pl.BlockSpec((8, _BK), lambda i, c: (i, c))],
        out_specs=pl.BlockSpec((8, 1, 1, 128), lambda i, c: (i, c, 0, 0)),
        out_shape=jax.ShapeDtypeStruct((_R, _NBK, 1, 128), jnp.float32),
    )(logits)
    cm = cm3[:, :, 0, :].reshape(_R, _NP)
    perm = pl.pallas_call(
        _select_body,
        out_shape=jax.ShapeDtypeStruct((_R, _K), jnp.int32),
    )(cm)
    v = pl.pallas_call(
        _thresh_body,
        grid=(_R,),
        in_specs=[
            pl.BlockSpec(memory_space=pltpu.SMEM),
            pl.BlockSpec(memory_space=pl.ANY),
            pl.BlockSpec((8, _CW), lambda i: (i // 8, _NC)),
        ],
        out_specs=pl.BlockSpec((1, 1, 128), lambda i: (i, 0, 0)),
        out_shape=jax.ShapeDtypeStruct((_R, 1, 128), jnp.float32),
        scratch_shapes=[
            pltpu.VMEM((8 * _K, _CW), jnp.float32),
            pltpu.VMEM((_K + 8, _CW), jnp.float32),
            pltpu.VMEM((_K + 8, _CW), jnp.int32),
            pltpu.SemaphoreType.DMA,
        ],
    )(perm, logits, logits)
    out = pl.pallas_call(
        _gate_body,
        grid=(_R // 8, _NBK),
        in_specs=[
            pl.BlockSpec((8, _BK), lambda i, c: (i, c)),
            pl.BlockSpec((8, 1, 128), lambda i, c: (i, 0, 0)),
        ],
        out_specs=pl.BlockSpec((8, _BK), lambda i, c: (i, c)),
        out_shape=jax.ShapeDtypeStruct((_R, _W), jnp.float32),
    )(logits, v)
    return out


# speedup vs baseline: 3.2332x; 1.4964x over previous
import jax
import jax.numpy as jnp
from jax.experimental import pallas as pl
from jax.experimental.pallas import tpu as pltpu

_K = 64
_INV_TAU = 10.0
_R = 32
_W = 1_000_000
_CW = 1024
_NC = _W // _CW
_TAIL = _W - _NC * _CW
_CPB = 128
_BK = _CPB * _CW
_NBK = 8
_NP = _NBK * _CPB
_MIN32 = -2147483648


def _monotone_key(x):
    b = jax.lax.bitcast_convert_type(x, jnp.int32)
    return jnp.where(b >= 0, b, jnp.int32(_MIN32) - b)


def _chunkmax_body(x_ref, cm_ref):
    mx = [jnp.max(x_ref[:, pl.ds(j * _CW, _CW)], axis=1, keepdims=True)
          for j in range(_CPB)]
    cm_ref[:, 0, 0, :] = jnp.concatenate(mx, axis=1)


def _select_body(cm_ref, perm_ref):
    p_iota = jax.lax.broadcasted_iota(jnp.int32, (_R, _NP), 1)
    valid = p_iota < _NC
    keys = jnp.where(valid, _monotone_key(cm_ref[...]), jnp.int32(_MIN32))
    cnt = jnp.sum((keys >= 0).astype(jnp.int32), axis=1, keepdims=True)
    t = jnp.where(cnt >= _K, jnp.int32(0), jnp.int32(_MIN32))
    for b in range(30, -1, -1):
        cand_t = t + jnp.int32(1 << b)
        cnt = jnp.sum((keys >= cand_t).astype(jnp.int32), axis=1,
                      keepdims=True)
        t = jnp.where(cnt >= _K, cand_t, t)
    tri_r = jax.lax.broadcasted_iota(jnp.int32, (_NP, _NP), 0)
    tri_c = jax.lax.broadcasted_iota(jnp.int32, (_NP, _NP), 1)
    le = (tri_r <= tri_c).astype(jnp.float32)
    above = (keys > t)
    q = jnp.sum(above.astype(jnp.float32), axis=1, keepdims=True)
    tie = (keys == t)
    tie_rank = jnp.dot(tie.astype(jnp.float32), le,
                       preferred_element_type=jnp.float32)
    sel = above | (tie & (tie_rank <= (_K - q)))
    rank = jnp.dot(sel.astype(jnp.float32), le,
                   preferred_element_type=jnp.float32).astype(jnp.int32) - 1
    picked = jnp.where(sel, rank, jnp.int32(-1))
    for s in range(_K):
        perm_ref[:, pl.ds(s, 1)] = jnp.sum(
            jnp.where(picked == s, p_iota, 0), axis=1, keepdims=True)


_NS = _K + 8


def _thresh_body(perm_ref, x_ref, tail_ref, v_ref, gbuf_ref, cand_ref,
                 key_ref, sem):
    lane3 = jax.lax.broadcasted_iota(jnp.int32, (_R, 1, _CW), 2)
    cand_ref[:, pl.ds(_K, 1), :] = jnp.where(
        lane3 < _TAIL, tail_ref[...].reshape(_R, 1, _CW), -jnp.inf)
    cand_ref[:, pl.ds(_K + 1, 7), :] = jnp.full((_R, 7, _CW), -jnp.inf,
                                                jnp.float32)

    def row_body(r, carry):
        rg = r // 8
        rm = jax.lax.rem(r, 8)
        copies = []
        for s in range(_K):
            off = perm_ref[r, s] * _CW
            cp = pltpu.make_async_copy(
                x_ref.at[pl.ds(rg * 8, 8), pl.ds(off, _CW)],
                gbuf_ref.at[pl.ds(s * 8, 8), :], sem)
            cp.start()
            copies.append(cp)
        for cp in copies:
            cp.wait()
        for s in range(_K):
            cand_ref[pl.ds(r, 1), pl.ds(s, 1), :] = (
                gbuf_ref[pl.ds(s * 8 + rm, 1), :].reshape(1, 1, _CW))
        return carry

    jax.lax.fori_loop(0, _R, row_body, jnp.int32(0))

    key_ref[...] = _monotone_key(cand_ref[...])

    def count_ge(tt):
        return jnp.sum((key_ref[...] >= tt).astype(jnp.int32), axis=(1, 2),
                       keepdims=True)

    t = jnp.where(count_ge(jnp.int32(0)) >= _K, jnp.int32(0),
                  jnp.int32(_MIN32))
    for b in range(30, -1, -1):
        cand_t = t + jnp.int32(1 << b)
        t = jnp.where(count_ge(cand_t) >= _K, cand_t, t)
    v_bits = jnp.where(t >= 0, t, jnp.int32(_MIN32) - t)
    v = jax.lax.bitcast_convert_type(v_bits, jnp.float32)
    v_ref[...] = jnp.broadcast_to(v, (_R, 1, 128))


def _gate_body(x_ref, v_ref, o_ref):
    vv = v_ref[:, 0, pl.ds(0, 1)]
    xs = x_ref[...]
    z = (vv - xs) * jnp.float32(_INV_TAU)
    o_ref[...] = xs / (1.0 + jnp.exp(z))


def kernel(logits):
    cm3 = pl.pallas_call(
        _chunkmax_body,
        grid=(_R // 8, _NBK),
        in_specs=[pl.BlockSpec((8, _BK), lambda i, c: (i, c))],
        out_specs=pl.BlockSpec((8, 1, 1, 128), lambda i, c: (i, c, 0, 0)),
        out_shape=jax.ShapeDtypeStruct((_R, _NBK, 1, 128), jnp.float32),
    )(logits)
    cm = cm3[:, :, 0, :].reshape(_R, _NP)
    perm = pl.pallas_call(
        _select_body,
        out_shape=jax.ShapeDtypeStruct((_R, _K), jnp.int32),
    )(cm)
    v = pl.pallas_call(
        _thresh_body,
        grid=(1,),
        in_specs=[
            pl.BlockSpec(memory_space=pltpu.SMEM),
            pl.BlockSpec(memory_space=pl.ANY),
            pl.BlockSpec((_R, _CW), lambda i: (0, _NC)),
        ],
        out_specs=pl.BlockSpec((_R, 1, 128), lambda i: (0, 0, 0)),
        out_shape=jax.ShapeDtypeStruct((_R, 1, 128), jnp.float32),
        scratch_shapes=[
            pltpu.VMEM((8 * _K, _CW), jnp.float32),
            pltpu.VMEM((_R, _NS, _CW), jnp.float32),
            pltpu.VMEM((_R, _NS, _CW), jnp.int32),
            pltpu.SemaphoreType.DMA,
        ],
    )(perm, logits, logits)
    gbw = 2 * _BK
    out = pl.pallas_call(
        _gate_body,
        grid=(_R // 8, (_W + gbw - 1) // gbw),
        in_specs=[
            pl.BlockSpec((8, gbw), lambda i, c: (i, c)),
            pl.BlockSpec((8, 1, 128), lambda i, c: (i, 0, 0)),
        ],
        out_specs=pl.BlockSpec((8, gbw), lambda i, c: (i, c)),
        out_shape=jax.ShapeDtypeStruct((_R, _W), jnp.float32),
    )(logits, v)
    return out


# speedup vs baseline: 3.2888x; 1.0172x over previous
import jax
import jax.numpy as jnp
from jax.experimental import pallas as pl
from jax.experimental.pallas import tpu as pltpu

_K = 64
_INV_TAU = 10.0
_R = 32
_W = 1_000_000
_CW = 1024
_NC = _W // _CW
_TAIL = _W - _NC * _CW
_CPB = 128
_BK = _CPB * _CW
_NBK = 8
_NP = _NBK * _CPB
_MIN32 = -2147483648


def _monotone_key(x):
    b = jax.lax.bitcast_convert_type(x, jnp.int32)
    return jnp.where(b >= 0, b, jnp.int32(_MIN32) - b)


def _chunkmax_body(x_ref, cm_ref):
    mx = [jnp.max(x_ref[:, pl.ds(j * _CW, _CW)], axis=1, keepdims=True)
          for j in range(_CPB)]
    cm_ref[:, 0, 0, :] = jnp.concatenate(mx, axis=1)


def _select_body(cm_ref, perm_ref):
    p_iota = jax.lax.broadcasted_iota(jnp.int32, (_R, _NP), 1)
    valid = p_iota < _NC
    keys = jnp.where(valid, _monotone_key(cm_ref[...]), jnp.int32(_MIN32))
    cnt = jnp.sum((keys >= 0).astype(jnp.int32), axis=1, keepdims=True)
    t = jnp.where(cnt >= _K, jnp.int32(0), jnp.int32(_MIN32))
    for b in range(30, -1, -1):
        cand_t = t + jnp.int32(1 << b)
        cnt = jnp.sum((keys >= cand_t).astype(jnp.int32), axis=1,
                      keepdims=True)
        t = jnp.where(cnt >= _K, cand_t, t)
    tri_r = jax.lax.broadcasted_iota(jnp.int32, (_NP, _NP), 0)
    tri_c = jax.lax.broadcasted_iota(jnp.int32, (_NP, _NP), 1)
    le = (tri_r <= tri_c).astype(jnp.float32)
    above = (keys > t)
    q = jnp.sum(above.astype(jnp.float32), axis=1, keepdims=True)
    tie = (keys == t)
    tie_rank = jnp.dot(tie.astype(jnp.float32), le,
                       preferred_element_type=jnp.float32)
    sel = above | (tie & (tie_rank <= (_K - q)))
    rank = jnp.dot(sel.astype(jnp.float32), le,
                   preferred_element_type=jnp.float32).astype(jnp.int32) - 1
    picked = jnp.where(sel, rank, jnp.int32(-1))
    for s in range(_K):
        perm_ref[:, pl.ds(s, 1)] = jnp.sum(
            jnp.where(picked == s, p_iota, 0), axis=1, keepdims=True)


_NS = _K + 8


def _thresh_body(perm_ref, x_ref, tail_ref, v_ref, gbuf_ref, cand_ref,
                 key_ref, sem):
    lane3 = jax.lax.broadcasted_iota(jnp.int32, (_R, 1, _CW), 2)
    cand_ref[:, pl.ds(_K, 1), :] = jnp.where(
        lane3 < _TAIL, tail_ref[...].reshape(_R, 1, _CW), -jnp.inf)
    cand_ref[:, pl.ds(_K + 1, 7), :] = jnp.full((_R, 7, _CW), -jnp.inf,
                                                jnp.float32)

    def row_body(r, carry):
        rg = r // 8
        rm = jax.lax.rem(r, 8)
        copies = []
        for s in range(_K):
            off = perm_ref[r, s] * _CW
            cp = pltpu.make_async_copy(
                x_ref.at[pl.ds(rg * 8, 8), pl.ds(off, _CW)],
                gbuf_ref.at[pl.ds(s * 8, 8), :], sem)
            cp.start()
            copies.append(cp)
        for s in range(_K):
            copies[s].wait()
            cand_ref[pl.ds(r, 1), pl.ds(s, 1), :] = (
                gbuf_ref[pl.ds(s * 8 + rm, 1), :].reshape(1, 1, _CW))
        return carry

    jax.lax.fori_loop(0, _R, row_body, jnp.int32(0))

    key_ref[...] = _monotone_key(cand_ref[...])

    def count_ge(tt):
        return jnp.sum((key_ref[...] >= tt).astype(jnp.int32), axis=(1, 2),
                       keepdims=True)

    t = jnp.where(count_ge(jnp.int32(0)) >= _K, jnp.int32(0),
                  jnp.int32(_MIN32))
    for b in range(30, -1, -1):
        cand_t = t + jnp.int32(1 << b)
        t = jnp.where(count_ge(cand_t) >= _K, cand_t, t)
    v_bits = jnp.where(t >= 0, t, jnp.int32(_MIN32) - t)
    v = jax.lax.bitcast_convert_type(v_bits, jnp.float32)
    v_ref[...] = jnp.broadcast_to(v, (_R, 1, 128))


def _gate_body(x_ref, v_ref, o_ref):
    vv = v_ref[:, 0, pl.ds(0, 1)]
    xs = x_ref[...]
    z = (vv - xs) * jnp.float32(_INV_TAU)
    o_ref[...] = xs / (1.0 + jnp.exp(z))


def kernel(logits):
    cm3 = pl.pallas_call(
        _chunkmax_body,
        grid=(_R // 8, _NBK),
        in_specs=[pl.BlockSpec((8, _BK), lambda i, c: (i, c))],
        out_specs=pl.BlockSpec((8, 1, 1, 128), lambda i, c: (i, c, 0, 0)),
        out_shape=jax.ShapeDtypeStruct((_R, _NBK, 1, 128), jnp.float32),
    )(logits)
    cm = cm3[:, :, 0, :].reshape(_R, _NP)
    perm = pl.pallas_call(
        _select_body,
        out_shape=jax.ShapeDtypeStruct((_R, _K), jnp.int32),
    )(cm)
    v = pl.pallas_call(
        _thresh_body,
        grid=(1,),
        in_specs=[
            pl.BlockSpec(memory_space=pltpu.SMEM),
            pl.BlockSpec(memory_space=pl.ANY),
            pl.BlockSpec((_R, _CW), lambda i: (0, _NC)),
        ],
        out_specs=pl.BlockSpec((_R, 1, 128), lambda i: (0, 0, 0)),
        out_shape=jax.ShapeDtypeStruct((_R, 1, 128), jnp.float32),
        scratch_shapes=[
            pltpu.VMEM((8 * _K, _CW), jnp.float32),
            pltpu.VMEM((_R, _NS, _CW), jnp.float32),
            pltpu.VMEM((_R, _NS, _CW), jnp.int32),
            pltpu.SemaphoreType.DMA,
        ],
    )(perm, logits, logits)
    gbw = 2 * _BK
    out = pl.pallas_call(
        _gate_body,
        grid=(_R // 8, (_W + gbw - 1) // gbw),
        in_specs=[
            pl.BlockSpec((8, gbw), lambda i, c: (i, c)),
            pl.BlockSpec((8, 1, 128), lambda i, c: (i, 0, 0)),
        ],
        out_specs=pl.BlockSpec((8, gbw), lambda i, c: (i, c)),
        out_shape=jax.ShapeDtypeStruct((_R, _W), jnp.float32),
    )(logits, v)
    return out
